# baseline (speedup 1.0000x reference)
import functools

import jax
import jax.numpy as jnp
from jax import lax
from jax.experimental import pallas as pl
from jax.experimental.pallas import tpu as pltpu
from jax.experimental.pallas import tpu_sc as plsc

N = 10000
E = 320000
M = 20000
IN = 128
OUT = 128
DEGE = 0.25
DEGV = 0.25
SCALE = DEGE * DEGV

H = OUT // 2
NC, NS, L = 2, 16, 16

EPT = E // NS
K = 200
NCHUNK = EPT // K

MB = 1248
M_REM = M - 15 * MB
NB = 624
N_REM = N - 15 * NB
CBUF = 1280 + L

_MESH = plsc.VectorSubcoreMesh(core_axis_name="c", subcore_axis_name="s",
                               num_cores=NC, num_subcores=NS)
_PARAMS = pltpu.CompilerParams(use_tc_tiling_on_sc=False)


def _matmul(X, W):
    BN = 400

    def body(x_ref, w_ref, o0_ref, o1_ref):
        r = lax.dot_general(x_ref[...], w_ref[...], (((1,), (1,)), ((), ())),
                            preferred_element_type=jnp.float32)
        o0_ref[...] = r[:, :H]
        o1_ref[...] = r[:, H:]

    return pl.pallas_call(
        body,
        grid=(N // BN,),
        in_specs=[
            pl.BlockSpec((BN, IN), lambda i: (i, 0)),
            pl.BlockSpec((OUT, IN), lambda i: (0, 0)),
        ],
        out_specs=[
            pl.BlockSpec((BN, H), lambda i: (i, 0)),
            pl.BlockSpec((BN, H), lambda i: (i, 0)),
        ],
        out_shape=[
            jax.ShapeDtypeStruct((N, H), jnp.float32),
            jax.ShapeDtypeStruct((N, H), jnp.float32),
        ],
    )(X, W)


def _zero_rows_buf(rows, k):
    @pl.loop(0, k)
    def _(i):
        for j in range(H // L):
            rows[i, pl.ds(j * L, L)] = jnp.zeros((L,), jnp.float32)


def _chunks(total, cap):
    return [(lo, min(cap, total - lo)) for lo in range(0, total, cap)]


def _make_pair_pipeline(s, c, vtx_hbm, edg_hbm, gsrc0, gsrc1, k, nchunk,
                        rows, idx, scatter, post=None):
    def idx_issue(g, slot):
        vs, es, sem = idx[slot]
        base = s * EPT + g * k
        pltpu.async_copy(vtx_hbm.at[pl.ds(base, k)], vs, sem)
        pltpu.async_copy(edg_hbm.at[pl.ds(base, k)], es, sem)

    def idx_wait(slot):
        vs, es, sem = idx[slot]
        pltpu.make_async_copy(vtx_hbm.at[pl.ds(0, k)], vs, sem).wait()
        pltpu.make_async_copy(edg_hbm.at[pl.ds(0, k)], es, sem).wait()

    def gather_issue(slot, rb):
        vs, es, _ = idx[slot]
        rbuf, sem = rows[rb]

        @pl.when(c == 0)
        def _():
            pltpu.async_copy(gsrc0(vs, es), rbuf, sem)

        @pl.when(c == 1)
        def _():
            pltpu.async_copy(gsrc1(vs, es), rbuf, sem)

    def commit(slot, rb):
        rbuf, sem = rows[rb]
        vs, es, _ = idx[slot]
        pltpu.make_async_copy(gsrc0(vs, es), rbuf, sem).wait()
        scatter(rbuf, vs, es)
        if post is not None:
            post[0](es, post[2][slot])

    idx_issue(0, 0)
    idx_issue(1, 1)
    idx_wait(0)
    gather_issue(0, 0)

    @pl.loop(0, nchunk, step=4)
    def _(g):
        for q in range(4):
            if q < 3:
                idx_wait(q + 1)
                gather_issue(q + 1, (q + 1) % 2)
            else:
                @pl.when(g + 4 < nchunk)
                def _():
                    idx_wait(0)
                    gather_issue(0, 0)

            commit(q, q % 2)

            slot2 = (q + 2) % 4
            @pl.when(g + q + 2 < nchunk)
            def _():
                if post is not None:
                    @pl.when(g + q >= 2)
                    def _():
                        post[1](idx[slot2][1], post[2][slot2])
                idx_issue(g + q + 2, slot2)

    if post is not None:
        for slot in range(4):
            post[1](idx[slot][1], post[2][slot])


def _sc_body(xp0_hbm, xp1_hbm, vtx_hbm, edg_hbm, out_hbm, xe0_hbm, xe1_hbm,
             acc, cnt, rows0, rows1,
             vidx0, eidx0, vidx1, eidx1, vidx2, eidx2, vidx3, eidx3,
             ones, cbuf, zbuf, semg0, semg1, semi0, semi1, semi2, semi3,
             semo0, semo1, semo2, semo3):
    c = lax.axis_index("c")
    s = lax.axis_index("s")
    last = s == NS - 1
    bm = s * MB
    bn = s * NB
    rows = ((rows0, semg0), (rows1, semg1))
    idx = ((vidx0, eidx0, semi0), (vidx1, eidx1, semi1),
           (vidx2, eidx2, semi2), (vidx3, eidx3, semi3))
    semo = (semo0, semo1, semo2, semo3)

    _zero_rows_buf(zbuf, K)

    @pl.loop(0, CBUF // L)
    def _(i):
        cbuf[pl.ds(i * L, L)] = jnp.zeros((L,), jnp.float32)

    @pl.loop(0, K // L)
    def _(i):
        ones[pl.ds(i * L, L)] = jnp.ones((L,), jnp.float32)

    if K % L:
        ones[pl.ds(K - L, L)] = jnp.ones((L,), jnp.float32)

    def _zero_acc(nrows, base):
        ch = _chunks(nrows, K)
        for lo, sz in ch:
            pltpu.async_copy(zbuf.at[pl.ds(0, sz)],
                             acc.at[pl.ds(base + lo, sz)], semg0)
        for lo, sz in ch:
            pltpu.make_async_copy(zbuf.at[pl.ds(0, sz)],
                                  acc.at[pl.ds(base + lo, sz)], semg0).wait()

    def _zero_cnt(nrows, base):
        for lo, sz in _chunks(nrows, CBUF):
            pltpu.sync_copy(cbuf.at[pl.ds(0, sz)],
                            cnt.at[pl.ds(base + lo, sz)])

    _zero_acc(MB, bm)
    _zero_cnt(MB, bm)

    @pl.when(last)
    def _():
        _zero_acc(M_REM - MB, 16 * MB)
        _zero_cnt(M_REM - MB, 16 * MB)

    plsc.subcore_barrier()

    def scatter_a(rbuf, vs, es):
        pltpu.sync_copy(rbuf, acc.at[es], add=True)

    def ones_issue(es, sem):
        pltpu.make_async_copy(ones, cnt.at[es], sem).start(add=True)

    def ones_wait(es, sem):
        pltpu.make_async_copy(ones, cnt.at[es], sem).wait()

    _make_pair_pipeline(
        s, c, vtx_hbm, edg_hbm,
        lambda vs, es: xp0_hbm.at[vs], lambda vs, es: xp1_hbm.at[vs],
        K, NCHUNK, rows, idx, scatter_a,
        post=(ones_issue, ones_wait, semo))

    plsc.subcore_barrier()

    def _store_xe(rbuf, row_base, sz, sem):
        @pl.when(c == 0)
        def _():
            pltpu.async_copy(rbuf.at[pl.ds(0, sz)],
                             xe0_hbm.at[pl.ds(row_base, sz)], sem)

        @pl.when(c == 1)
        def _():
            pltpu.async_copy(rbuf.at[pl.ds(0, sz)],
                             xe1_hbm.at[pl.ds(row_base, sz)], sem)

    def _scale_chunk(rbuf, lo, sz):
        @pl.loop(0, sz)
        def _(r):
            scale = cbuf[pl.ds(r + lo, L)][0]
            for j in range(H // L):
                rbuf[r, pl.ds(j * L, L)] = rbuf[r, pl.ds(j * L, L)] * scale

    def _scale_rows(row_base, total):
        pltpu.sync_copy(cnt.at[pl.ds(row_base, total)],
                        cbuf.at[pl.ds(0, total)])

        @pl.loop(0, (total + L - 1) // L)
        def _(i):
            cv = cbuf[pl.ds(i * L, L)]
            cbuf[pl.ds(i * L, L)] = SCALE / jnp.maximum(cv, 1.0)

        ch = _chunks(total, K)
        semL = (semi0, semi1)
        semS = (semi2, semi3)
        bufs = (rows0, rows1)
        n = len(ch)
        for i, (lo, sz) in enumerate(ch):
            b = i % 2
            if i == 0:
                pltpu.sync_copy(acc.at[pl.ds(row_base + lo, sz)],
                                bufs[b].at[pl.ds(0, sz)])
            else:
                pltpu.make_async_copy(
                    acc.at[pl.ds(row_base + lo, sz)],
                    bufs[b].at[pl.ds(0, sz)], semL[b]).wait()
            pltpu.async_copy(zbuf.at[pl.ds(0, sz)],
                             acc.at[pl.ds(row_base + lo, sz)], semg1)
            if i + 1 < n:
                nlo, nsz = ch[i + 1]
                nb = (i + 1) % 2
                if i >= 1:
                    slo, ssz = ch[i - 1]
                    pltpu.make_async_copy(
                        bufs[nb].at[pl.ds(0, ssz)],
                        xe0_hbm.at[pl.ds(row_base + slo, ssz)],
                        semS[nb]).wait()
                pltpu.async_copy(acc.at[pl.ds(row_base + nlo, nsz)],
                                 bufs[nb].at[pl.ds(0, nsz)], semL[nb])
            _scale_chunk(bufs[b], lo, sz)
            _store_xe(bufs[b], row_base + lo, sz, semS[b])
        for i in range(max(0, n - 2), n):
            slo, ssz = ch[i]
            pltpu.make_async_copy(bufs[i % 2].at[pl.ds(0, ssz)],
                                  xe0_hbm.at[pl.ds(row_base + slo, ssz)],
                                  semS[i % 2]).wait()
        for lo, sz in ch:
            pltpu.make_async_copy(zbuf.at[pl.ds(0, sz)],
                                  acc.at[pl.ds(row_base + lo, sz)],
                                  semg1).wait()

    _scale_rows(bm, MB)

    @pl.when(last)
    def _():
        _scale_rows(16 * MB, M_REM - MB)

    plsc.subcore_barrier()

    def scatter_b(rbuf, vs, es):
        pltpu.sync_copy(rbuf, acc.at[vs], add=True)

    _make_pair_pipeline(
        s, c, vtx_hbm, edg_hbm,
        lambda vs, es: xe0_hbm.at[es], lambda vs, es: xe1_hbm.at[es],
        K, NCHUNK, rows, idx, scatter_b)

    plsc.subcore_barrier()

    def _writeout(row_base, nrows):
        ch = _chunks(nrows, K)
        semS = (semi2, semi3)
        bufs = (rows0, rows1)
        for i, (lo, sz) in enumerate(ch):
            b = i % 2
            if i >= 2:
                slo, ssz = ch[i - 2]
                pltpu.make_async_copy(
                    bufs[b].at[pl.ds(0, ssz)],
                    out_hbm.at[pl.ds(row_base + slo, ssz), pl.ds(0, H)],
                    semS[b]).wait()
            pltpu.sync_copy(acc.at[pl.ds(row_base + lo, sz)],
                            bufs[b].at[pl.ds(0, sz)])

            @pl.when(c == 0)
            def _():
                pltpu.async_copy(
                    bufs[b].at[pl.ds(0, sz)],
                    out_hbm.at[pl.ds(row_base + lo, sz), pl.ds(0, H)],
                    semS[b])

            @pl.when(c == 1)
            def _():
                pltpu.async_copy(
                    bufs[b].at[pl.ds(0, sz)],
                    out_hbm.at[pl.ds(row_base + lo, sz), pl.ds(H, H)],
                    semS[b])
        n = len(ch)
        for i in range(max(0, n - 2), n):
            slo, ssz = ch[i]
            pltpu.make_async_copy(
                bufs[i % 2].at[pl.ds(0, ssz)],
                out_hbm.at[pl.ds(row_base + slo, ssz), pl.ds(0, H)],
                semS[i % 2]).wait()

    _writeout(bn, NB)

    @pl.when(last)
    def _():
        _writeout(16 * NB, N_REM - NB)


_sc = functools.partial(
    pl.kernel,
    out_type=[
        jax.ShapeDtypeStruct((N, OUT), jnp.float32),
        jax.ShapeDtypeStruct((M, H), jnp.float32),
        jax.ShapeDtypeStruct((M, H), jnp.float32),
    ],
    mesh=_MESH,
    compiler_params=_PARAMS,
    scratch_types=[
        pltpu.VMEM_SHARED((M, H), jnp.float32),
        pltpu.VMEM_SHARED((M,), jnp.float32),
        pltpu.VMEM((K, H), jnp.float32),
        pltpu.VMEM((K, H), jnp.float32),
        pltpu.VMEM((K,), jnp.int32),
        pltpu.VMEM((K,), jnp.int32),
        pltpu.VMEM((K,), jnp.int32),
        pltpu.VMEM((K,), jnp.int32),
        pltpu.VMEM((K,), jnp.int32),
        pltpu.VMEM((K,), jnp.int32),
        pltpu.VMEM((K,), jnp.int32),
        pltpu.VMEM((K,), jnp.int32),
        pltpu.VMEM((K,), jnp.float32),
        pltpu.VMEM((CBUF,), jnp.float32),
        pltpu.VMEM((K, H), jnp.float32),
        pltpu.SemaphoreType.DMA,
        pltpu.SemaphoreType.DMA,
        pltpu.SemaphoreType.DMA,
        pltpu.SemaphoreType.DMA,
        pltpu.SemaphoreType.DMA,
        pltpu.SemaphoreType.DMA,
        pltpu.SemaphoreType.DMA,
        pltpu.SemaphoreType.DMA,
        pltpu.SemaphoreType.DMA,
        pltpu.SemaphoreType.DMA,
    ],
)(_sc_body)


@jax.jit
def kernel(X, vertex, edges, W):
    xp0, xp1 = _matmul(X, W)
    out, _, _ = _sc(xp0, xp1, vertex, edges)
    return out


# speedup vs baseline: 1.1672x; 1.1672x over previous
import functools

import jax
import jax.numpy as jnp
from jax import lax
from jax.experimental import pallas as pl
from jax.experimental.pallas import tpu as pltpu
from jax.experimental.pallas import tpu_sc as plsc

N = 10000
E = 320000
M = 20000
IN = 128
OUT = 128
DEGE = 0.25
DEGV = 0.25
SCALE = DEGE * DEGV

H = OUT // 2
NC, NS, L = 2, 16, 16

EPT = E // NS
K = 200
NCHUNK = EPT // K

MB = 1248
M_REM = M - 15 * MB
NB = 624
N_REM = N - 15 * NB
CBUF = 1280 + L

_MESH = plsc.VectorSubcoreMesh(core_axis_name="c", subcore_axis_name="s",
                               num_cores=NC, num_subcores=NS)
_PARAMS = pltpu.CompilerParams(use_tc_tiling_on_sc=False)


def _matmul(X, W):
    BN = 400

    def body(x_ref, w_ref, o0_ref, o1_ref):
        r = lax.dot_general(x_ref[...], w_ref[...], (((1,), (1,)), ((), ())),
                            preferred_element_type=jnp.float32)
        o0_ref[...] = r[:, :H]
        o1_ref[...] = r[:, H:]

    return pl.pallas_call(
        body,
        grid=(N // BN,),
        in_specs=[
            pl.BlockSpec((BN, IN), lambda i: (i, 0)),
            pl.BlockSpec((OUT, IN), lambda i: (0, 0)),
        ],
        out_specs=[
            pl.BlockSpec((BN, H), lambda i: (i, 0)),
            pl.BlockSpec((BN, H), lambda i: (i, 0)),
        ],
        out_shape=[
            jax.ShapeDtypeStruct((N, H), jnp.float32),
            jax.ShapeDtypeStruct((N, H), jnp.float32),
        ],
    )(X, W)


def _zero_rows_buf(rows, k):
    @pl.loop(0, k)
    def _(i):
        for j in range(H // L):
            rows[i, pl.ds(j * L, L)] = jnp.zeros((L,), jnp.float32)


def _chunks(total, cap):
    return [(lo, min(cap, total - lo)) for lo in range(0, total, cap)]


def _make_pair_pipeline(s, c, vtx_hbm, edg_hbm, gsrc0, gsrc1, k, nchunk,
                        rows, idx, scatter, post=None):
    def idx_issue(g, slot):
        vs, es, sem = idx[slot]
        base = s * EPT + g * k
        pltpu.async_copy(vtx_hbm.at[pl.ds(base, k)], vs, sem)
        pltpu.async_copy(edg_hbm.at[pl.ds(base, k)], es, sem)

    def idx_wait(slot):
        vs, es, sem = idx[slot]
        pltpu.make_async_copy(vtx_hbm.at[pl.ds(0, k)], vs, sem).wait()
        pltpu.make_async_copy(edg_hbm.at[pl.ds(0, k)], es, sem).wait()

    def gather_issue(slot, rb):
        vs, es, _ = idx[slot]
        rbuf, sem = rows[rb]

        @pl.when(c == 0)
        def _():
            pltpu.async_copy(gsrc0(vs, es), rbuf, sem)

        @pl.when(c == 1)
        def _():
            pltpu.async_copy(gsrc1(vs, es), rbuf, sem)

    def commit(slot, rb):
        rbuf, sem = rows[rb]
        vs, es, _ = idx[slot]
        pltpu.make_async_copy(gsrc0(vs, es), rbuf, sem).wait()
        scatter(rbuf, vs, es)
        if post is not None:
            post(es)

    for q in range(4):
        idx_issue(q, q)
    idx_wait(0)
    gather_issue(0, 0)

    @pl.loop(0, nchunk, step=4)
    def _(g):
        for q in range(4):
            if q < 3:
                idx_wait(q + 1)
                gather_issue(q + 1, (q + 1) % 2)
            else:
                @pl.when(g + 4 < nchunk)
                def _():
                    idx_wait(0)
                    gather_issue(0, 0)

            commit(q, q % 2)

            @pl.when(g + q + 4 < nchunk)
            def _():
                idx_issue(g + q + 4, q)


def _sc_body(xp0_hbm, xp1_hbm, vtx_hbm, edg_hbm, out_hbm, xe0_hbm, xe1_hbm,
             acc, cnt, rows0, rows1,
             vidx0, eidx0, vidx1, eidx1, vidx2, eidx2, vidx3, eidx3,
             ones, cbuf, zbuf, semg0, semg1, semi0, semi1, semi2, semi3):
    c = lax.axis_index("c")
    s = lax.axis_index("s")
    last = s == NS - 1
    bm = s * MB
    bn = s * NB
    rows = ((rows0, semg0), (rows1, semg1))
    idx = ((vidx0, eidx0, semi0), (vidx1, eidx1, semi1),
           (vidx2, eidx2, semi2), (vidx3, eidx3, semi3))
    _zero_rows_buf(zbuf, K)

    @pl.loop(0, CBUF // L)
    def _(i):
        cbuf[pl.ds(i * L, L)] = jnp.zeros((L,), jnp.float32)

    @pl.loop(0, K // L)
    def _(i):
        ones[pl.ds(i * L, L)] = jnp.ones((L,), jnp.float32)

    if K % L:
        ones[pl.ds(K - L, L)] = jnp.ones((L,), jnp.float32)

    def _zero_acc(nrows, base):
        ch = _chunks(nrows, K)
        for lo, sz in ch:
            pltpu.async_copy(zbuf.at[pl.ds(0, sz)],
                             acc.at[pl.ds(base + lo, sz)], semg0)
        for lo, sz in ch:
            pltpu.make_async_copy(zbuf.at[pl.ds(0, sz)],
                                  acc.at[pl.ds(base + lo, sz)], semg0).wait()

    def _zero_cnt(nrows, base):
        for lo, sz in _chunks(nrows, CBUF):
            pltpu.sync_copy(cbuf.at[pl.ds(0, sz)],
                            cnt.at[pl.ds(base + lo, sz)])

    _zero_acc(MB, bm)
    _zero_cnt(MB, bm)

    @pl.when(last)
    def _():
        _zero_acc(M_REM - MB, 16 * MB)
        _zero_cnt(M_REM - MB, 16 * MB)

    plsc.subcore_barrier()

    def scatter_a(rbuf, vs, es):
        pltpu.sync_copy(rbuf, acc.at[es], add=True)

    def ones_post(es):
        pltpu.sync_copy(ones, cnt.at[es], add=True)

    _make_pair_pipeline(
        s, c, vtx_hbm, edg_hbm,
        lambda vs, es: xp0_hbm.at[vs], lambda vs, es: xp1_hbm.at[vs],
        K, NCHUNK, rows, idx, scatter_a, post=ones_post)

    plsc.subcore_barrier()

    def _store_xe(rbuf, row_base, sz, sem):
        @pl.when(c == 0)
        def _():
            pltpu.async_copy(rbuf.at[pl.ds(0, sz)],
                             xe0_hbm.at[pl.ds(row_base, sz)], sem)

        @pl.when(c == 1)
        def _():
            pltpu.async_copy(rbuf.at[pl.ds(0, sz)],
                             xe1_hbm.at[pl.ds(row_base, sz)], sem)

    def _scale_chunk(rbuf, lo, sz):
        @pl.loop(0, sz)
        def _(r):
            scale = cbuf[pl.ds(r + lo, L)][0]
            for j in range(H // L):
                rbuf[r, pl.ds(j * L, L)] = rbuf[r, pl.ds(j * L, L)] * scale

    def _scale_rows(row_base, total):
        pltpu.sync_copy(cnt.at[pl.ds(row_base, total)],
                        cbuf.at[pl.ds(0, total)])

        @pl.loop(0, (total + L - 1) // L)
        def _(i):
            cv = cbuf[pl.ds(i * L, L)]
            cbuf[pl.ds(i * L, L)] = SCALE / jnp.maximum(cv, 1.0)

        ch = _chunks(total, K)
        semL = (semi0, semi1)
        semS = (semi2, semi3)
        bufs = (rows0, rows1)
        n = len(ch)
        for i, (lo, sz) in enumerate(ch):
            b = i % 2
            if i == 0:
                pltpu.sync_copy(acc.at[pl.ds(row_base + lo, sz)],
                                bufs[b].at[pl.ds(0, sz)])
            else:
                pltpu.make_async_copy(
                    acc.at[pl.ds(row_base + lo, sz)],
                    bufs[b].at[pl.ds(0, sz)], semL[b]).wait()
            pltpu.async_copy(zbuf.at[pl.ds(0, sz)],
                             acc.at[pl.ds(row_base + lo, sz)], semg1)
            if i + 1 < n:
                nlo, nsz = ch[i + 1]
                nb = (i + 1) % 2
                if i >= 1:
                    slo, ssz = ch[i - 1]
                    pltpu.make_async_copy(
                        bufs[nb].at[pl.ds(0, ssz)],
                        xe0_hbm.at[pl.ds(row_base + slo, ssz)],
                        semS[nb]).wait()
                pltpu.async_copy(acc.at[pl.ds(row_base + nlo, nsz)],
                                 bufs[nb].at[pl.ds(0, nsz)], semL[nb])
            _scale_chunk(bufs[b], lo, sz)
            _store_xe(bufs[b], row_base + lo, sz, semS[b])
        for i in range(max(0, n - 2), n):
            slo, ssz = ch[i]
            pltpu.make_async_copy(bufs[i % 2].at[pl.ds(0, ssz)],
                                  xe0_hbm.at[pl.ds(row_base + slo, ssz)],
                                  semS[i % 2]).wait()
        for lo, sz in ch:
            pltpu.make_async_copy(zbuf.at[pl.ds(0, sz)],
                                  acc.at[pl.ds(row_base + lo, sz)],
                                  semg1).wait()

    _scale_rows(bm, MB)

    @pl.when(last)
    def _():
        _scale_rows(16 * MB, M_REM - MB)

    plsc.subcore_barrier()

    def scatter_b(rbuf, vs, es):
        pltpu.sync_copy(rbuf, acc.at[vs], add=True)

    _make_pair_pipeline(
        s, c, vtx_hbm, edg_hbm,
        lambda vs, es: xe0_hbm.at[es], lambda vs, es: xe1_hbm.at[es],
        K, NCHUNK, rows, idx, scatter_b)

    plsc.subcore_barrier()

    def _writeout(row_base, nrows):
        ch = _chunks(nrows, K)
        semS = (semi2, semi3)
        bufs = (rows0, rows1)
        for i, (lo, sz) in enumerate(ch):
            b = i % 2
            if i >= 2:
                slo, ssz = ch[i - 2]
                pltpu.make_async_copy(
                    bufs[b].at[pl.ds(0, ssz)],
                    out_hbm.at[pl.ds(row_base + slo, ssz), pl.ds(0, H)],
                    semS[b]).wait()
            pltpu.sync_copy(acc.at[pl.ds(row_base + lo, sz)],
                            bufs[b].at[pl.ds(0, sz)])

            @pl.when(c == 0)
            def _():
                pltpu.async_copy(
                    bufs[b].at[pl.ds(0, sz)],
                    out_hbm.at[pl.ds(row_base + lo, sz), pl.ds(0, H)],
                    semS[b])

            @pl.when(c == 1)
            def _():
                pltpu.async_copy(
                    bufs[b].at[pl.ds(0, sz)],
                    out_hbm.at[pl.ds(row_base + lo, sz), pl.ds(H, H)],
                    semS[b])
        n = len(ch)
        for i in range(max(0, n - 2), n):
            slo, ssz = ch[i]
            pltpu.make_async_copy(
                bufs[i % 2].at[pl.ds(0, ssz)],
                out_hbm.at[pl.ds(row_base + slo, ssz), pl.ds(0, H)],
                semS[i % 2]).wait()

    _writeout(bn, NB)

    @pl.when(last)
    def _():
        _writeout(16 * NB, N_REM - NB)


_sc = functools.partial(
    pl.kernel,
    out_type=[
        jax.ShapeDtypeStruct((N, OUT), jnp.float32),
        jax.ShapeDtypeStruct((M, H), jnp.float32),
        jax.ShapeDtypeStruct((M, H), jnp.float32),
    ],
    mesh=_MESH,
    compiler_params=_PARAMS,
    scratch_types=[
        pltpu.VMEM_SHARED((M, H), jnp.float32),
        pltpu.VMEM_SHARED((M,), jnp.float32),
        pltpu.VMEM((K, H), jnp.float32),
        pltpu.VMEM((K, H), jnp.float32),
        pltpu.VMEM((K,), jnp.int32),
        pltpu.VMEM((K,), jnp.int32),
        pltpu.VMEM((K,), jnp.int32),
        pltpu.VMEM((K,), jnp.int32),
        pltpu.VMEM((K,), jnp.int32),
        pltpu.VMEM((K,), jnp.int32),
        pltpu.VMEM((K,), jnp.int32),
        pltpu.VMEM((K,), jnp.int32),
        pltpu.VMEM((K,), jnp.float32),
        pltpu.VMEM((CBUF,), jnp.float32),
        pltpu.VMEM((K, H), jnp.float32),
        pltpu.SemaphoreType.DMA,
        pltpu.SemaphoreType.DMA,
        pltpu.SemaphoreType.DMA,
        pltpu.SemaphoreType.DMA,
        pltpu.SemaphoreType.DMA,
        pltpu.SemaphoreType.DMA,
    ],
)(_sc_body)


@jax.jit
def kernel(X, vertex, edges, W):
    xp0, xp1 = _matmul(X, W)
    out, _, _ = _sc(xp0, xp1, vertex, edges)
    return out
